# baseline (speedup 1.0000x reference)
import jax
import jax.numpy as jnp
from jax import lax
from jax.experimental import pallas as pl
from jax.experimental.pallas import tpu as pltpu
from jax.experimental.pallas import tpu_sc as plsc

VOCAB = 100000
MAXLEN = 200
EMBED = 64
BATCH = 1024

NC = 2
NS = 16
NW = NC * NS
LANES = 16

ROWS = BATCH * MAXLEN
R_PER_W = ROWS // NW
CHUNK = MAXLEN
N_CHUNKS = R_PER_W // CHUNK
Q = EMBED // LANES
NB = 3


def _emb_kernel(idx_hbm, tok_hbm, pos_hbm, out_hbm,
                idx_v, rows_v, pos_v, sg0, sg1, sg2, so0, so1, so2):
    semg = (sg0, sg1, sg2)
    semo = (so0, so1, so2)
    wid = lax.axis_index("s") * NC + lax.axis_index("c")
    base = wid * R_PER_W

    pltpu.sync_copy(pos_hbm, pos_v)

    def start_gather(c):
        b = c % NB
        s = base + c * CHUNK
        pltpu.sync_copy(idx_hbm.at[pl.ds(s, CHUNK)], idx_v.at[b])
        return pltpu.async_copy(tok_hbm.at[idx_v.at[b]], rows_v.at[b], semg[b])

    pending_g = {0: start_gather(0)}
    pending_o = {}
    for c in range(N_CHUNKS):
        b = c % NB
        nxt = c + 1
        if nxt < N_CHUNKS:
            if nxt - NB >= 0:
                pending_o.pop(nxt - NB).wait()
            pending_g[nxt] = start_gather(nxt)
        pending_g.pop(c).wait()

        @plsc.parallel_loop(0, CHUNK, 1, unroll=8)
        def _(r, b=b):
            for q in range(Q):
                plsc.addupdate(rows_v.at[b, r, pl.ds(q * LANES, LANES)],
                               pos_v[r, pl.ds(q * LANES, LANES)])

        s = base + c * CHUNK
        pending_o[c] = pltpu.async_copy(
            rows_v.at[b], out_hbm.at[pl.ds(s, CHUNK)], semo[b])

    for c in sorted(pending_o):
        pending_o.pop(c).wait()


@jax.jit
def _run(idx_flat, token_table, pos_table):
    mesh = plsc.VectorSubcoreMesh(core_axis_name="c", subcore_axis_name="s")
    f = pl.kernel(
        _emb_kernel,
        out_type=jax.ShapeDtypeStruct((ROWS, EMBED), jnp.float32),
        mesh=mesh,
        scratch_types=[
            pltpu.VMEM((NB, CHUNK), jnp.int32),
            pltpu.VMEM((NB, CHUNK, EMBED), jnp.float32),
            pltpu.VMEM((MAXLEN, EMBED), jnp.float32),
        ] + [pltpu.SemaphoreType.DMA] * (2 * NB),
        compiler_params=pltpu.CompilerParams(use_tc_tiling_on_sc=False),
    )
    return f(idx_flat, token_table, pos_table)


def kernel(inputs, token_table, pos_table):
    idx_flat = inputs.reshape(-1).astype(jnp.int32)
    out = _run(idx_flat, token_table, pos_table)
    return out.reshape(BATCH, MAXLEN, EMBED)
